# baseline (speedup 1.0000x reference)
import functools

import jax
import jax.numpy as jnp
from jax import lax
from jax.experimental import pallas as pl
from jax.experimental.pallas import tpu as pltpu
from jax.experimental.pallas import tpu_sc as plsc

OUT_H = 7
OUT_W = 7
SPATIAL_SCALE = 0.125
N, C, H, W = 2, 256, 128, 128
K = 1000
KP = 1024
G = OUT_H * OUT_W
P = G * KP
CH = 32
LANES = 16

_SLOT_OF_CH = [32 * (c // 32) + (c % 32) // 2 + 16 * (c % 2) for c in range(256)]


def _tc_index_kernel(rt_ref, idx_ref, w_ref):
    bi = rt_ref[0:1, :].astype(jnp.int32)
    cx = rt_ref[1:2, :]
    cy = rt_ref[2:3, :]
    rw = rt_ref[3:4, :]
    rh = rt_ref[4:5, :]
    ct = rt_ref[5:6, :]
    st = rt_ref[6:7, :]
    gi = lax.broadcasted_iota(jnp.int32, (G, KP), 0)
    gxf = (gi % OUT_W).astype(jnp.float32)
    gyf = (gi // OUT_W).astype(jnp.float32)
    gx = (gxf + 0.5) / OUT_W - 0.5
    gy = (gyf + 0.5) / OUT_H - 0.5
    gxw = gx * rw
    gyh = gy * rh
    ix = gxw * ct - gyh * st + cx - 0.5
    iy = gxw * st + gyh * ct + cy - 0.5
    x0 = jnp.floor(ix)
    y0 = jnp.floor(iy)
    wx1 = ix - x0
    wx0 = 1.0 - wx1
    wy1 = iy - y0
    wy0 = 1.0 - wy1
    bbase = bi * (H * W)
    corners = (
        (x0, y0, wx0 * wy0),
        (x0 + 1.0, y0, wx1 * wy0),
        (x0, y0 + 1.0, wx0 * wy1),
        (x0 + 1.0, y0 + 1.0, wx1 * wy1),
    )
    for j, (xc, yc, wj) in enumerate(corners):
        valid = (xc >= 0.0) & (xc <= W - 1.0) & (yc >= 0.0) & (yc <= H - 1.0)
        xi = jnp.clip(xc, 0.0, W - 1.0).astype(jnp.int32)
        yi = jnp.clip(yc, 0.0, H - 1.0).astype(jnp.int32)
        idx_ref[j] = bbase + yi * W + xi
        w_ref[j] = jnp.where(valid, wj, 0.0)


_tc_index = pl.pallas_call(
    _tc_index_kernel,
    out_shape=(
        jax.ShapeDtypeStruct((4, G, KP), jnp.int32),
        jax.ShapeDtypeStruct((4, G, KP), jnp.float32),
    ),
)


def _make_sc_gather(nw):
    per_w = P // nw
    n_chunks = per_w // CH
    mesh = plsc.VectorSubcoreMesh(core_axis_name="c", subcore_axis_name="s")

    @functools.partial(
        pl.kernel,
        mesh=mesh,
        out_type=jax.ShapeDtypeStruct((P, C), jnp.float32),
        scratch_types=[
            pltpu.VMEM((n_chunks, 1, 4 * CH), jnp.int32),
            pltpu.VMEM((2, 4 * CH, LANES), jnp.float32),
            pltpu.VMEM((2, 4 * CH, C // 2), jnp.uint32),
            pltpu.VMEM((2, CH, C), jnp.float32),
            pltpu.SemaphoreType.DMA,
            pltpu.SemaphoreType.DMA,
            pltpu.SemaphoreType.DMA,
            pltpu.SemaphoreType.DMA,
            pltpu.SemaphoreType.DMA,
            pltpu.SemaphoreType.DMA,
        ],
    )
    def sc_fn(feats_hbm, idx_hbm, w_hbm, out_hbm, idx_all, w_v, rows_v, out_v,
              gs0, gs1, ws0, ws1, os0, os1):
        gsems = (gs0, gs1)
        wsems = (ws0, ws1)
        osems = (os0, os1)
        wid = lax.axis_index("s") * 2 + lax.axis_index("c")
        cbase = wid * n_chunks
        pbase = wid * per_w
        pltpu.sync_copy(idx_hbm.at[wid], idx_all)

        def start(ck, b):
            pltpu.async_copy(w_hbm.at[cbase + ck], w_v.at[b], wsems[b])
            pltpu.async_copy(feats_hbm.at[idx_all.at[ck, 0]], rows_v.at[b], gsems[b])

        def compute(ck, b):
            pltpu.make_async_copy(w_hbm.at[cbase + ck], w_v.at[b], wsems[b]).wait()
            pltpu.make_async_copy(
                feats_hbm.at[idx_all.at[ck, 0]], rows_v.at[b], gsems[b]).wait()

            @pl.when(ck >= 2)
            def _():
                pltpu.make_async_copy(
                    out_v.at[b], out_hbm.at[pl.ds(0, CH)], osems[b]).wait()

            def pbody(p, c2):
                wb = (w_v[b, p, :], w_v[b, CH + p, :],
                      w_v[b, 2 * CH + p, :], w_v[b, 3 * CH + p, :])
                for g in range(C // (2 * LANES)):
                    sl = pl.ds(g * LANES, LANES)
                    acc_a = jnp.zeros((LANES,), jnp.float32)
                    acc_b = jnp.zeros((LANES,), jnp.float32)
                    for j in range(4):
                        u = rows_v[b, j * CH + p, sl]
                        va = lax.bitcast_convert_type(u << 16, jnp.float32)
                        vb = lax.bitcast_convert_type(u & jnp.uint32(0xFFFF0000), jnp.float32)
                        acc_a = acc_a + va * wb[j]
                        acc_b = acc_b + vb * wb[j]
                    out_v[b, p, pl.ds(g * 2 * LANES, LANES)] = acc_a
                    out_v[b, p, pl.ds(g * 2 * LANES + LANES, LANES)] = acc_b
                return c2

            lax.fori_loop(0, CH, pbody, 0)
            pltpu.async_copy(
                out_v.at[b], out_hbm.at[pl.ds(pbase + ck * CH, CH)], osems[b])

        start(0, 0)

        def pair(g, carry):
            ck = 2 * g
            start(ck + 1, 1)
            compute(ck, 0)
            start(ck + 2, 0)
            compute(ck + 1, 1)
            return carry

        lax.fori_loop(0, (n_chunks - 1) // 2, pair, 0)
        compute(n_chunks - 1, 0)
        pltpu.make_async_copy(out_v.at[0], out_hbm.at[pl.ds(0, CH)], osems[0]).wait()
        pltpu.make_async_copy(out_v.at[1], out_hbm.at[pl.ds(0, CH)], osems[1]).wait()

    return sc_fn


@functools.cache
def _sc_gather_cached():
    return _make_sc_gather(32)


def kernel(features, rois):
    feats_flat = jnp.transpose(features, (0, 2, 3, 1)).reshape(N * H * W, C)
    feats_flat = jax.lax.bitcast_convert_type(
        feats_flat.astype(jnp.bfloat16).reshape(N * H * W, C // 2, 2),
        jnp.uint32)
    th = rois[:, 5] * SPATIAL_SCALE
    rt = jnp.stack(
        [
            rois[:, 0],
            rois[:, 1] * SPATIAL_SCALE,
            rois[:, 2] * SPATIAL_SCALE,
            rois[:, 3] * SPATIAL_SCALE,
            rois[:, 4] * SPATIAL_SCALE,
            jnp.cos(th),
            jnp.sin(th),
            jnp.zeros_like(th),
        ],
        axis=0,
    )
    rt = jnp.pad(rt, ((0, 0), (0, KP - K)))
    idx4, w4 = _tc_index(rt)
    n_chunks = P // CH // 32
    idx_sc = (
        idx4.reshape(4, P // CH, CH).transpose(1, 0, 2)
        .reshape(32, n_chunks, 1, 4 * CH)
    )
    w_sc = w4.reshape(4, P // CH, CH).transpose(1, 0, 2).reshape(P // CH, 4 * CH)
    w_sc = jnp.broadcast_to(w_sc[:, :, None], (P // CH, 4 * CH, LANES))
    out2 = _sc_gather_cached()(feats_flat, idx_sc, w_sc)
    out = out2.reshape(G, KP, C)[:, :K]
    out = out[:, :, jnp.asarray(_SLOT_OF_CH)]
    return out.transpose(1, 2, 0).reshape(K, C, OUT_H, OUT_W)


# speedup vs baseline: 1.3576x; 1.3576x over previous
import functools

import jax
import jax.numpy as jnp
from jax import lax
from jax.experimental import pallas as pl
from jax.experimental.pallas import tpu as pltpu
from jax.experimental.pallas import tpu_sc as plsc

OUT_H = 7
OUT_W = 7
SPATIAL_SCALE = 0.125
N, C, H, W = 2, 256, 128, 128
K = 1000
KP = 1024
G = OUT_H * OUT_W
P = G * KP
CH = 32
LANES = 16


def _tc_index_kernel(rt_ref, idx_ref, w_ref):
    bi = rt_ref[0:1, :].astype(jnp.int32)
    cx = rt_ref[1:2, :]
    cy = rt_ref[2:3, :]
    rw = rt_ref[3:4, :]
    rh = rt_ref[4:5, :]
    ct = rt_ref[5:6, :]
    st = rt_ref[6:7, :]
    gi = lax.broadcasted_iota(jnp.int32, (G, KP), 0)
    gxf = (gi % OUT_W).astype(jnp.float32)
    gyf = (gi // OUT_W).astype(jnp.float32)
    gx = (gxf + 0.5) / OUT_W - 0.5
    gy = (gyf + 0.5) / OUT_H - 0.5
    gxw = gx * rw
    gyh = gy * rh
    ix = gxw * ct - gyh * st + cx - 0.5
    iy = gxw * st + gyh * ct + cy - 0.5
    x0 = jnp.floor(ix)
    y0 = jnp.floor(iy)
    wx1 = ix - x0
    wx0 = 1.0 - wx1
    wy1 = iy - y0
    wy0 = 1.0 - wy1
    bbase = bi * (H * W)
    corners = (
        (x0, y0, wx0 * wy0),
        (x0 + 1.0, y0, wx1 * wy0),
        (x0, y0 + 1.0, wx0 * wy1),
        (x0 + 1.0, y0 + 1.0, wx1 * wy1),
    )
    for j, (xc, yc, wj) in enumerate(corners):
        valid = (xc >= 0.0) & (xc <= W - 1.0) & (yc >= 0.0) & (yc <= H - 1.0)
        xi = jnp.clip(xc, 0.0, W - 1.0).astype(jnp.int32)
        yi = jnp.clip(yc, 0.0, H - 1.0).astype(jnp.int32)
        idx_ref[j] = bbase + yi * W + xi
        w_ref[j] = jnp.where(valid, wj, 0.0)


_tc_index = pl.pallas_call(
    _tc_index_kernel,
    out_shape=(
        jax.ShapeDtypeStruct((4, G, KP), jnp.int32),
        jax.ShapeDtypeStruct((4, G, KP), jnp.float32),
    ),
)


def _make_sc_gather(nw):
    per_w = P // nw
    n_chunks = per_w // CH
    mesh = plsc.VectorSubcoreMesh(core_axis_name="c", subcore_axis_name="s")

    @functools.partial(
        pl.kernel,
        mesh=mesh,
        out_type=jax.ShapeDtypeStruct((P, C), jnp.float32),
        scratch_types=[
            pltpu.VMEM((n_chunks, 1, 4 * CH), jnp.int32),
            pltpu.VMEM((2, 4 * CH, LANES), jnp.float32),
            pltpu.VMEM((2, 4 * CH, C // 2), jnp.uint32),
            pltpu.VMEM((2, CH, C), jnp.float32),
            pltpu.SemaphoreType.DMA,
            pltpu.SemaphoreType.DMA,
            pltpu.SemaphoreType.DMA,
            pltpu.SemaphoreType.DMA,
            pltpu.SemaphoreType.DMA,
            pltpu.SemaphoreType.DMA,
        ],
    )
    def sc_fn(feats_hbm, idx_hbm, w_hbm, out_hbm, idx_all, w_v, rows_v, out_v,
              gs0, gs1, ws0, ws1, os0, os1):
        gsems = (gs0, gs1)
        wsems = (ws0, ws1)
        osems = (os0, os1)
        wid = lax.axis_index("s") * 2 + lax.axis_index("c")
        cbase = wid * n_chunks
        pbase = wid * per_w
        pltpu.sync_copy(idx_hbm.at[wid], idx_all)

        def start(ck, b):
            pltpu.async_copy(w_hbm.at[cbase + ck], w_v.at[b], wsems[b])
            pltpu.async_copy(feats_hbm.at[idx_all.at[ck, 0]], rows_v.at[b], gsems[b])

        def compute(ck, b):
            pltpu.make_async_copy(w_hbm.at[cbase + ck], w_v.at[b], wsems[b]).wait()
            pltpu.make_async_copy(
                feats_hbm.at[idx_all.at[ck, 0]], rows_v.at[b], gsems[b]).wait()

            @pl.when(ck >= 2)
            def _():
                pltpu.make_async_copy(
                    out_v.at[b], out_hbm.at[pl.ds(0, CH)], osems[b]).wait()

            def pbody(p, c2):
                wb = (w_v[b, p, :], w_v[b, CH + p, :],
                      w_v[b, 2 * CH + p, :], w_v[b, 3 * CH + p, :])
                for g in range(C // (2 * LANES)):
                    sl = pl.ds(g * LANES, LANES)
                    acc_a = jnp.zeros((LANES,), jnp.float32)
                    acc_b = jnp.zeros((LANES,), jnp.float32)
                    for j in range(4):
                        u = rows_v[b, j * CH + p, sl]
                        va = lax.bitcast_convert_type(u << 16, jnp.float32)
                        vb = lax.bitcast_convert_type(u & jnp.uint32(0xFFFF0000), jnp.float32)
                        acc_a = acc_a + va * wb[j]
                        acc_b = acc_b + vb * wb[j]
                    out_v[b, p, pl.ds(g * LANES, LANES)] = acc_a
                    out_v[b, p, pl.ds(C // 2 + g * LANES, LANES)] = acc_b
                return c2

            lax.fori_loop(0, CH, pbody, 0)
            pltpu.async_copy(
                out_v.at[b], out_hbm.at[pl.ds(pbase + ck * CH, CH)], osems[b])

        start(0, 0)

        def pair(g, carry):
            ck = 2 * g
            start(ck + 1, 1)
            compute(ck, 0)
            start(ck + 2, 0)
            compute(ck + 1, 1)
            return carry

        lax.fori_loop(0, (n_chunks - 1) // 2, pair, 0)
        compute(n_chunks - 1, 0)
        pltpu.make_async_copy(out_v.at[0], out_hbm.at[pl.ds(0, CH)], osems[0]).wait()
        pltpu.make_async_copy(out_v.at[1], out_hbm.at[pl.ds(0, CH)], osems[1]).wait()

    return sc_fn


@functools.cache
def _sc_gather_cached():
    return _make_sc_gather(32)


def kernel(features, rois):
    fb = features.astype(jnp.bfloat16).reshape(N, 2, C // 2, H * W)
    fb = fb.transpose(0, 3, 2, 1)
    feats_flat = jax.lax.bitcast_convert_type(fb, jnp.uint32)
    feats_flat = feats_flat.reshape(N * H * W, C // 2)
    th = rois[:, 5] * SPATIAL_SCALE
    rt = jnp.stack(
        [
            rois[:, 0],
            rois[:, 1] * SPATIAL_SCALE,
            rois[:, 2] * SPATIAL_SCALE,
            rois[:, 3] * SPATIAL_SCALE,
            rois[:, 4] * SPATIAL_SCALE,
            jnp.cos(th),
            jnp.sin(th),
            jnp.zeros_like(th),
        ],
        axis=0,
    )
    rt = jnp.pad(rt, ((0, 0), (0, KP - K)))
    idx4, w4 = _tc_index(rt)
    n_chunks = P // CH // 32
    idx_sc = (
        idx4.reshape(4, P // CH, CH).transpose(1, 0, 2)
        .reshape(32, n_chunks, 1, 4 * CH)
    )
    w_sc = w4.reshape(4, P // CH, CH).transpose(1, 0, 2).reshape(P // CH, 4 * CH)
    w_sc = jnp.broadcast_to(w_sc[:, :, None], (P // CH, 4 * CH, LANES))
    out2 = _sc_gather_cached()(feats_flat, idx_sc, w_sc)
    out = out2.reshape(G, KP, C)[:, :K]
    return out.transpose(1, 2, 0).reshape(K, C, OUT_H, OUT_W)
